# baseline (speedup 1.0000x reference)
import functools

import jax
import jax.numpy as jnp
from jax import lax
from jax.experimental import pallas as pl
from jax.experimental.pallas import tpu as pltpu
from jax.experimental.pallas import tpu_sc as plsc

_B = 16384
_D = 64
_NEG = 10
_MARGIN = 0.8
_NC = 2
_NS = 16
_NW = _NC * _NS
_CU = 32
_LANES = 16


def _rsqrt(x):
    x = jnp.maximum(x, 1e-24)
    xi = plsc.bitcast(x, jnp.int32)
    yi = jnp.full(x.shape, 0x5F3759DF, jnp.int32) - (xi >> 1)
    y = plsc.bitcast(yi, jnp.float32)
    for _ in range(3):
        y = y * (1.5 - 0.5 * x * y * y)
    return y


def _build(b, d, neg, margin, interpret=False):
    nch = b // (_NW * _CU)
    ngrp = _CU // _LANES
    ncu = _CU * neg
    mesh = plsc.VectorSubcoreMesh(core_axis_name="c", subcore_axis_name="s")

    @functools.partial(
        pl.kernel,
        out_type=jax.ShapeDtypeStruct((_NW, _LANES), jnp.float32),
        mesh=mesh,
        scratch_types=[
            pltpu.VMEM((_CU,), jnp.int32),
            pltpu.VMEM((_CU,), jnp.int32),
            pltpu.VMEM((ncu,), jnp.int32),
            [pltpu.VMEM((_CU, d + 1), jnp.float32)] * 2,
            [pltpu.VMEM((_CU, d + 1), jnp.float32)] * 2,
            [pltpu.VMEM((ncu, d + 1), jnp.float32)] * 2,
            pltpu.VMEM((_LANES,), jnp.float32),
            [pltpu.SemaphoreType.DMA] * 2,
        ],
        compiler_params=pltpu.CompilerParams(
            needs_layout_passes=False, use_tc_tiling_on_sc=True),
        interpret=interpret,
    )
    def sc_loss(u2d, i2d, n2d, uemb, iemb, out, idxu_v, idxi_v, idxn_v,
                ubufs, ibufs, nbufs, accv, sems):
        wid = lax.axis_index("s") * _NC + lax.axis_index("c")
        iot = lax.iota(jnp.int32, _LANES)

        def prefetch(c, s):
            gc = wid * nch + c
            pltpu.sync_copy(u2d.at[gc], idxu_v)
            pltpu.sync_copy(i2d.at[gc], idxi_v)
            pltpu.sync_copy(n2d.at[gc], idxn_v)

            def enq(n_rows, idx_v, dst, src):
                def body(g, _):
                    ids = idx_v[pl.ds(g * _LANES, _LANES)]
                    for j in range(_LANES):
                        pltpu.async_copy(
                            src.at[ids[j]],
                            dst.at[g * _LANES + j, pl.ds(0, d)], sems[s])
                    return 0
                lax.fori_loop(0, n_rows // _LANES, body, 0)

            enq(_CU, idxu_v, ubufs[s], uemb)
            enq(_CU, idxi_v, ibufs[s], iemb)
            enq(ncu, idxn_v, nbufs[s], iemb)

        def drain(s):
            def mk(n_rows, dst, src):
                def body(j, _):
                    pltpu.make_async_copy(
                        src.at[0], dst.at[j, pl.ds(0, d)], sems[s]).wait()
                    return 0
                lax.fori_loop(0, n_rows, body, 0)
            mk(_CU, ubufs[s], uemb)
            mk(_CU, ibufs[s], iemb)
            mk(ncu, nbufs[s], iemb)

        def compute(s, acc):
            ubuf, ibuf, nbuf = ubufs[s], ibufs[s], nbufs[s]

            def group_body(g, acc):
                lane = g * _LANES + iot
                nrows = [lane * neg + k for k in range(neg)]
                z = jnp.zeros((_LANES,), jnp.float32)
                init = (z, z, z) + tuple(z for _ in range(2 * neg))

                def d_body(dd, carry):
                    u2, i2, dp = carry[0], carry[1], carry[2]
                    dv = jnp.full((_LANES,), dd, jnp.int32)
                    gu = plsc.load_gather(ubuf, [lane, dv])
                    gi = plsc.load_gather(ibuf, [lane, dv])
                    u2 = u2 + gu * gu
                    i2 = i2 + gi * gi
                    dp = dp + gu * gi
                    n2o, dno = [], []
                    for k in range(neg):
                        gn = plsc.load_gather(nbuf, [nrows[k], dv])
                        n2o.append(carry[3 + k] + gn * gn)
                        dno.append(carry[3 + neg + k] + gu * gn)
                    return (u2, i2, dp) + tuple(n2o) + tuple(dno)

                res = lax.fori_loop(0, d, d_body, init)
                ru = _rsqrt(res[0])
                ri = _rsqrt(res[1])
                predp = res[2] * ru * ri
                acc = acc + jnp.maximum(1.0 - predp, 0.0) * (1.0 / b)
                for k in range(neg):
                    rn = _rsqrt(res[3 + k])
                    predn = res[3 + neg + k] * ru * rn
                    acc = acc + jnp.maximum(predn - margin, 0.0)
                return acc

            return lax.fori_loop(0, ngrp, group_body, acc)

        prefetch(0, 0)
        acc = jnp.zeros((_LANES,), jnp.float32)

        def pipe_body(c2, acc):
            c = 2 * c2
            prefetch(c + 1, 1)
            drain(0)
            acc = compute(0, acc)

            @pl.when(c2 < nch // 2 - 1)
            def _():
                prefetch(c + 2, 0)
            drain(1)
            return compute(1, acc)

        acc = lax.fori_loop(0, nch // 2, pipe_body, acc)
        accv[...] = acc
        pltpu.sync_copy(accv, out.at[wid])

    def run(u, i, neg_idx, user_emb, item_emb):
        u2d = u.astype(jnp.int32).reshape(b // _CU, _CU)
        i2d = i.astype(jnp.int32).reshape(b // _CU, _CU)
        n2d = neg_idx.astype(jnp.int32).reshape(b // _CU, ncu)
        ue_b, ie_b = jax.lax.optimization_barrier((user_emb, item_emb))
        part = sc_loss(u2d, i2d, n2d, ue_b, ie_b)
        return jnp.sum(part)

    return run


_kernel_impl = None


def kernel(u, i, neg_idx, user_emb, item_emb):
    global _kernel_impl
    if _kernel_impl is None:
        _kernel_impl = jax.jit(_build(_B, _D, _NEG, _MARGIN))
    return _kernel_impl(u, i, neg_idx, user_emb, item_emb)


# speedup vs baseline: 1.0216x; 1.0216x over previous
import functools

import jax
import jax.numpy as jnp
from jax import lax
from jax.experimental import pallas as pl
from jax.experimental.pallas import tpu as pltpu
from jax.experimental.pallas import tpu_sc as plsc

_B = 16384
_D = 64
_NEG = 10
_MARGIN = 0.8
_NC = 2
_NS = 16
_NW = _NC * _NS
_CU = 32
_LANES = 16


def _rsqrt(x):
    x = jnp.maximum(x, 1e-24)
    xi = plsc.bitcast(x, jnp.int32)
    yi = jnp.full(x.shape, 0x5F3759DF, jnp.int32) - (xi >> 1)
    y = plsc.bitcast(yi, jnp.float32)
    for _ in range(3):
        y = y * (1.5 - 0.5 * x * y * y)
    return y


def _build(b, d, neg, margin, interpret=False):
    nch = b // (_NW * _CU)
    ngrp = _CU // _LANES
    ncu = _CU * neg
    mesh = plsc.VectorSubcoreMesh(core_axis_name="c", subcore_axis_name="s")

    @functools.partial(
        pl.kernel,
        out_type=jax.ShapeDtypeStruct((_NW, _LANES), jnp.float32),
        mesh=mesh,
        scratch_types=[
            pltpu.VMEM((_CU,), jnp.int32),
            pltpu.VMEM((_CU,), jnp.int32),
            pltpu.VMEM((ncu,), jnp.int32),
            [pltpu.VMEM((_CU, d), jnp.float32)] * 2,
            [pltpu.VMEM((_CU, d), jnp.float32)] * 2,
            [pltpu.VMEM((ncu, d), jnp.float32)] * 2,
            pltpu.VMEM((_LANES,), jnp.float32),
            [pltpu.SemaphoreType.DMA] * 2,
        ],
        compiler_params=pltpu.CompilerParams(
            needs_layout_passes=False, use_tc_tiling_on_sc=True),
        interpret=interpret,
    )
    def sc_loss(u2d, i2d, n2d, uemb, iemb, out, idxu_v, idxi_v, idxn_v,
                ubufs, ibufs, nbufs, accv, sems):
        wid = lax.axis_index("s") * _NC + lax.axis_index("c")
        iot = lax.iota(jnp.int32, _LANES)

        def prefetch(c, s):
            gc = wid * nch + c
            pltpu.sync_copy(u2d.at[gc], idxu_v)
            pltpu.sync_copy(i2d.at[gc], idxi_v)
            pltpu.sync_copy(n2d.at[gc], idxn_v)

            def enq(n_rows, idx_v, dst, src):
                def body(g, _):
                    ids = idx_v[pl.ds(g * _LANES, _LANES)]
                    for j in range(_LANES):
                        pltpu.async_copy(src.at[ids[j]],
                                         dst.at[g * _LANES + j], sems[s])
                    return 0
                lax.fori_loop(0, n_rows // _LANES, body, 0)

            enq(_CU, idxu_v, ubufs[s], uemb)
            enq(_CU, idxi_v, ibufs[s], iemb)
            enq(ncu, idxn_v, nbufs[s], iemb)

        def drain(s):
            pltpu.make_async_copy(uemb.at[pl.ds(0, _CU)], ubufs[s], sems[s]).wait()
            pltpu.make_async_copy(iemb.at[pl.ds(0, _CU)], ibufs[s], sems[s]).wait()
            pltpu.make_async_copy(iemb.at[pl.ds(0, ncu)], nbufs[s], sems[s]).wait()

        def compute(s, acc):
            ubuf, ibuf, nbuf = ubufs[s], ibufs[s], nbufs[s]

            def group_body(g, acc):
                lane = g * _LANES + iot
                nrows = [lane * neg + k for k in range(neg)]
                z = jnp.zeros((_LANES,), jnp.float32)
                init = (z, z, z) + tuple(z for _ in range(2 * neg))

                def d_body(dd, carry):
                    u2, i2, dp = carry[0], carry[1], carry[2]
                    dv = jnp.full((_LANES,), dd, jnp.int32)
                    gu = plsc.load_gather(ubuf, [lane, dv])
                    gi = plsc.load_gather(ibuf, [lane, dv])
                    u2 = u2 + gu * gu
                    i2 = i2 + gi * gi
                    dp = dp + gu * gi
                    n2o, dno = [], []
                    for k in range(neg):
                        gn = plsc.load_gather(nbuf, [nrows[k], dv])
                        n2o.append(carry[3 + k] + gn * gn)
                        dno.append(carry[3 + neg + k] + gu * gn)
                    return (u2, i2, dp) + tuple(n2o) + tuple(dno)

                res = lax.fori_loop(0, d, d_body, init)
                ru = _rsqrt(res[0])
                ri = _rsqrt(res[1])
                predp = res[2] * ru * ri
                acc = acc + jnp.maximum(1.0 - predp, 0.0) * (1.0 / b)
                for k in range(neg):
                    rn = _rsqrt(res[3 + k])
                    predn = res[3 + neg + k] * ru * rn
                    acc = acc + jnp.maximum(predn - margin, 0.0)
                return acc

            return lax.fori_loop(0, ngrp, group_body, acc)

        prefetch(0, 0)
        acc = jnp.zeros((_LANES,), jnp.float32)

        def pipe_body(c2, acc):
            c = 2 * c2
            prefetch(c + 1, 1)
            drain(0)
            acc = compute(0, acc)

            @pl.when(c2 < nch // 2 - 1)
            def _():
                prefetch(c + 2, 0)
            drain(1)
            return compute(1, acc)

        acc = lax.fori_loop(0, nch // 2, pipe_body, acc)
        accv[...] = acc
        pltpu.sync_copy(accv, out.at[wid])

    def run(u, i, neg_idx, user_emb, item_emb):
        u2d = u.astype(jnp.int32).reshape(b // _CU, _CU)
        i2d = i.astype(jnp.int32).reshape(b // _CU, _CU)
        n2d = neg_idx.astype(jnp.int32).reshape(b // _CU, ncu)
        part = sc_loss(u2d, i2d, n2d, user_emb, item_emb)
        return jnp.sum(part)

    return run


_kernel_impl = None


def kernel(u, i, neg_idx, user_emb, item_emb):
    global _kernel_impl
    if _kernel_impl is None:
        _kernel_impl = jax.jit(_build(_B, _D, _NEG, _MARGIN))
    return _kernel_impl(u, i, neg_idx, user_emb, item_emb)


# speedup vs baseline: 1.3753x; 1.3462x over previous
import functools

import jax
import jax.numpy as jnp
from jax import lax
from jax.experimental import pallas as pl
from jax.experimental.pallas import tpu as pltpu
from jax.experimental.pallas import tpu_sc as plsc

_B = 16384
_D = 64
_NEG = 10
_MARGIN = 0.8
_NC = 2
_NS = 16
_NW = _NC * _NS
_CU = 32
_LANES = 16


def _rsqrt(x):
    x = jnp.maximum(x, 1e-24)
    xi = plsc.bitcast(x, jnp.int32)
    yi = jnp.full(x.shape, 0x5F3759DF, jnp.int32) - (xi >> 1)
    y = plsc.bitcast(yi, jnp.float32)
    for _ in range(3):
        y = y * (1.5 - 0.5 * x * y * y)
    return y


def _build(b, d, neg, margin, interpret=False):
    nch = b // (_NW * _CU)
    ngrp = _CU // _LANES
    ncu = _CU * neg
    mesh = plsc.VectorSubcoreMesh(core_axis_name="c", subcore_axis_name="s")

    @functools.partial(
        pl.kernel,
        out_type=jax.ShapeDtypeStruct((_NW, _LANES), jnp.float32),
        mesh=mesh,
        scratch_types=[
            pltpu.VMEM((_CU,), jnp.int32),
            pltpu.VMEM((_CU,), jnp.int32),
            pltpu.VMEM((ncu,), jnp.int32),
            [pltpu.VMEM((_CU, d), jnp.float32)] * 2,
            [pltpu.VMEM((_CU, d), jnp.float32)] * 2,
            [pltpu.VMEM((ncu, d), jnp.float32)] * 2,
            pltpu.VMEM((_LANES,), jnp.float32),
            [pltpu.SemaphoreType.DMA] * 2,
        ],
        compiler_params=pltpu.CompilerParams(
            needs_layout_passes=False, use_tc_tiling_on_sc=True),
        interpret=interpret,
    )
    def sc_loss(u2d, i2d, n2d, uemb, iemb, out, idxu_v, idxi_v, idxn_v,
                ubufs, ibufs, nbufs, accv, sems):
        wid = lax.axis_index("s") * _NC + lax.axis_index("c")
        iot = lax.iota(jnp.int32, _LANES)

        def prefetch(c, s):
            gc = wid * nch + c
            pltpu.sync_copy(u2d.at[gc], idxu_v)
            pltpu.sync_copy(i2d.at[gc], idxi_v)
            pltpu.sync_copy(n2d.at[gc], idxn_v)

            def enq(n_rows, idx_v, dst, src):
                def body(g, _):
                    ids = idx_v[pl.ds(g * _LANES, _LANES)]
                    for j in range(_LANES):
                        pltpu.async_copy(src.at[0, ids[j]],
                                         dst.at[g * _LANES + j], sems[s])
                    return 0
                lax.fori_loop(0, n_rows // _LANES, body, 0)

            enq(_CU, idxu_v, ubufs[s], uemb)
            enq(_CU, idxi_v, ibufs[s], iemb)
            enq(ncu, idxn_v, nbufs[s], iemb)

        def drain(s):
            pltpu.make_async_copy(uemb.at[0, pl.ds(0, _CU)], ubufs[s], sems[s]).wait()
            pltpu.make_async_copy(iemb.at[0, pl.ds(0, _CU)], ibufs[s], sems[s]).wait()
            pltpu.make_async_copy(iemb.at[0, pl.ds(0, ncu)], nbufs[s], sems[s]).wait()

        def compute(s, acc):
            ubuf, ibuf, nbuf = ubufs[s], ibufs[s], nbufs[s]

            def group_body(g, acc):
                lane = g * _LANES + iot
                nrows = [lane * neg + k for k in range(neg)]
                z = jnp.zeros((_LANES,), jnp.float32)
                init = (z, z, z) + tuple(z for _ in range(2 * neg))

                def d_body(dd, carry):
                    u2, i2, dp = carry[0], carry[1], carry[2]
                    dv = jnp.full((_LANES,), dd, jnp.int32)
                    gu = plsc.load_gather(ubuf, [lane, dv])
                    gi = plsc.load_gather(ibuf, [lane, dv])
                    u2 = u2 + gu * gu
                    i2 = i2 + gi * gi
                    dp = dp + gu * gi
                    n2o, dno = [], []
                    for k in range(neg):
                        gn = plsc.load_gather(nbuf, [nrows[k], dv])
                        n2o.append(carry[3 + k] + gn * gn)
                        dno.append(carry[3 + neg + k] + gu * gn)
                    return (u2, i2, dp) + tuple(n2o) + tuple(dno)

                res = lax.fori_loop(0, d, d_body, init)
                ru = _rsqrt(res[0])
                ri = _rsqrt(res[1])
                predp = res[2] * ru * ri
                acc = acc + jnp.maximum(1.0 - predp, 0.0) * (1.0 / b)
                for k in range(neg):
                    rn = _rsqrt(res[3 + k])
                    predn = res[3 + neg + k] * ru * rn
                    acc = acc + jnp.maximum(predn - margin, 0.0)
                return acc

            return lax.fori_loop(0, ngrp, group_body, acc)

        prefetch(0, 0)
        acc = jnp.zeros((_LANES,), jnp.float32)

        def pipe_body(c2, acc):
            c = 2 * c2
            prefetch(c + 1, 1)
            drain(0)
            acc = compute(0, acc)

            @pl.when(c2 < nch // 2 - 1)
            def _():
                prefetch(c + 2, 0)
            drain(1)
            return compute(1, acc)

        acc = lax.fori_loop(0, nch // 2, pipe_body, acc)
        accv[...] = acc
        pltpu.sync_copy(accv, out.at[wid])

    def run(u, i, neg_idx, user_emb, item_emb):
        u2d = u.astype(jnp.int32).reshape(b // _CU, _CU)
        i2d = i.astype(jnp.int32).reshape(b // _CU, _CU)
        n2d = neg_idx.astype(jnp.int32).reshape(b // _CU, ncu)
        part = sc_loss(u2d, i2d, n2d, user_emb[None], item_emb[None])
        return jnp.sum(part)

    return run


_kernel_impl = None


def kernel(u, i, neg_idx, user_emb, item_emb):
    global _kernel_impl
    if _kernel_impl is None:
        _kernel_impl = jax.jit(_build(_B, _D, _NEG, _MARGIN))
    return _kernel_impl(u, i, neg_idx, user_emb, item_emb)
